# baseline (speedup 1.0000x reference)
import functools

import jax
import jax.numpy as jnp
from jax import lax
from jax.experimental import pallas as pl
from jax.experimental.pallas import tpu as pltpu
from jax.experimental.pallas import tpu_sc as plsc

NC, NS, L = 2, 16, 16
NW = NC * NS
ROWS, COLS = 128, 32768
RPW = ROWS // NW
UNROLL = 8
NVREG = COLS // L
NITER = NVREG // UNROLL

def _permute(v, idx):
    dnums = lax.GatherDimensionNumbers(
        offset_dims=(), collapsed_slice_dims=(0,), start_index_map=(0,)
    )
    return lax.gather(
        v, idx[:, None], dnums, (1,),
        mode=lax.GatherScatterMode.PROMISE_IN_BOUNDS,
    )


_mesh = plsc.VectorSubcoreMesh(
    core_axis_name="c", subcore_axis_name="s", num_cores=NC, num_subcores=NS
)


@functools.partial(
    pl.kernel,
    out_type=jax.ShapeDtypeStruct((ROWS,), jnp.int32),
    mesh=_mesh,
    scratch_types=[
        pltpu.VMEM((COLS,), jnp.float32),
        pltpu.VMEM((COLS,), jnp.float32),
        pltpu.VMEM((L,), jnp.int32),
        pltpu.VMEM((L,), jnp.int32),
        pltpu.SemaphoreType.DMA,
        pltpu.SemaphoreType.DMA,
        pltpu.SemaphoreType.DMA,
    ],
)
def _argmin_rows_sc(x_hbm, out_hbm, buf0, buf1, res_v, idx_v, sem0, sem1, sem2):
    wid = lax.axis_index("s") * NC + lax.axis_index("c")
    row0 = wid * RPW
    bufs = (buf0, buf1)
    sems = (sem0, sem1)
    lane = lax.iota(jnp.int32, 16)

    copies = [None] * RPW
    copies[0] = pltpu.async_copy(x_hbm.at[row0], buf0, sem0)

    res = jnp.zeros((L,), jnp.int32)
    for r in range(RPW):
        if r + 1 < RPW:
            copies[r + 1] = pltpu.async_copy(
                x_hbm.at[row0 + r + 1], bufs[(r + 1) % 2], sems[(r + 1) % 2]
            )
        copies[r].wait()
        buf = bufs[r % 2]

        def body(i, carry, buf=buf):
            ms, ids = list(carry[0]), list(carry[1])
            base = i * (UNROLL * L)
            for k in range(UNROLL):
                v = buf[pl.ds(base + k * L, L)]
                vi = jnp.full((L,), i * UNROLL + k, jnp.int32)
                pred = v < ms[k]
                ms[k] = jnp.where(pred, v, ms[k])
                ids[k] = jnp.where(pred, vi, ids[k])
            return tuple(ms), tuple(ids)

        inf = jnp.full((L,), jnp.inf, jnp.float32)
        zero = jnp.zeros((L,), jnp.int32)
        ms, ids = lax.fori_loop(
            0, NITER, body, ((inf,) * UNROLL, (zero,) * UNROLL)
        )

        m, g = ms[0], ids[0]
        for k in range(1, UNROLL):
            pred = (ms[k] < m) | ((ms[k] == m) & (ids[k] < g))
            m = jnp.where(pred, ms[k], m)
            g = jnp.where(pred, ids[k], g)

        e = g * L + lane
        for sh in (8, 4, 2, 1):
            perm = lane ^ sh
            mp = _permute(m, perm)
            ep = _permute(e, perm)
            pred = (mp < m) | ((mp == m) & (ep < e))
            m = jnp.where(pred, mp, m)
            e = jnp.where(pred, ep, e)
        if r < RPW - 1:
            res = jnp.where(lane == r, e, res)
        else:
            res = jnp.where(lane >= r, e, res)

    res_v[...] = res
    idx_v[...] = row0 + jnp.minimum(lane, RPW - 1)
    pltpu.async_copy(res_v, out_hbm.at[idx_v], sem2).wait()


def kernel(x):
    return _argmin_rows_sc(x)


# speedup vs baseline: 3.1780x; 3.1780x over previous
import functools

import jax
import jax.numpy as jnp
from jax import lax
from jax.experimental import pallas as pl
from jax.experimental.pallas import tpu as pltpu
from jax.experimental.pallas import tpu_sc as plsc

NC, NS, L = 2, 16, 16
NW = NC * NS
ROWS, COLS = 128, 32768
RPB = 8
HALF = COLS // 2
NCHUNK = 4
CW = HALF // NCHUNK
TPC = CW // 128
QPT = 128 // L


def _permute(v, idx):
    dnums = lax.GatherDimensionNumbers(
        offset_dims=(), collapsed_slice_dims=(0,), start_index_map=(0,)
    )
    return lax.gather(
        v, idx[:, None], dnums, (1,),
        mode=lax.GatherScatterMode.PROMISE_IN_BOUNDS,
    )


_mesh = plsc.VectorSubcoreMesh(
    core_axis_name="c", subcore_axis_name="s", num_cores=NC, num_subcores=NS
)


@functools.partial(
    pl.kernel,
    out_type=(
        jax.ShapeDtypeStruct((ROWS,), jnp.int32),
        jax.ShapeDtypeStruct((NW, L), jnp.float32),
        jax.ShapeDtypeStruct((NW, L), jnp.int32),
    ),
    mesh=_mesh,
    scratch_types=[
        pltpu.VMEM((2, RPB, CW), jnp.float32),
        pltpu.VMEM((L,), jnp.float32),
        pltpu.VMEM((L,), jnp.int32),
        pltpu.VMEM((L,), jnp.float32),
        pltpu.VMEM((L,), jnp.int32),
        pltpu.SemaphoreType.DMA,
        pltpu.SemaphoreType.DMA,
    ],
)
def _argmin_rows_sc(x_hbm, out_hbm, hm, he, buf, mv, ev, pmv, pev,
                    sem0, sem1):
    scid = lax.axis_index("c")
    sid = lax.axis_index("s")
    a = scid * (NS // 2) + sid // 2
    h = sid % 2
    r0 = a * RPB
    c0 = h * HALF
    lane = lax.iota(jnp.int32, 16)

    def chunk_src(c):
        return x_hbm.at[pl.ds(r0, RPB), pl.ds(c0 + c * CW, CW)]

    pltpu.async_copy(chunk_src(0), buf.at[0], sem0)
    pltpu.async_copy(chunk_src(1), buf.at[1], sem1)

    inf = jnp.full((L,), jnp.inf, jnp.float32)
    zero = jnp.zeros((L,), jnp.int32)

    def cbody(c, carry):
        ms, gs = list(carry[0]), list(carry[1])
        p = c % 2

        @pl.when(p == 0)
        def _():
            pltpu.make_async_copy(chunk_src(0), buf.at[0], sem0).wait()

        @pl.when(p == 1)
        def _():
            pltpu.make_async_copy(chunk_src(1), buf.at[1], sem1).wait()

        def jbody(j, carry2):
            ms2, gs2 = list(carry2[0]), list(carry2[1])
            for q in range(QPT):
                vi = c * (TPC * QPT) + j * QPT + q
                for s in range(RPB):
                    v = buf[p, s, pl.ds(j * 128 + q * L, L)]
                    pred = v < ms2[s]
                    ms2[s] = jnp.where(pred, v, ms2[s])
                    gs2[s] = jnp.where(pred, jnp.full((L,), vi, jnp.int32),
                                       gs2[s])
            return tuple(ms2), tuple(gs2)

        ms, gs = lax.fori_loop(0, TPC, jbody, (tuple(ms), tuple(gs)))

        @pl.when((c + 2 < NCHUNK) & (p == 0))
        def _():
            pltpu.async_copy(chunk_src(c + 2), buf.at[0], sem0)

        @pl.when((c + 2 < NCHUNK) & (p == 1))
        def _():
            pltpu.async_copy(chunk_src(c + 2), buf.at[1], sem1)

        return tuple(ms), tuple(gs)

    ms, gs = lax.fori_loop(
        0, NCHUNK, cbody, ((inf,) * RPB, (zero,) * RPB)
    )

    mres = jnp.full((L,), jnp.inf, jnp.float32)
    eres = jnp.zeros((L,), jnp.int32)
    for s in range(RPB):
        m = ms[s]
        e = (gs[s] * L + lane) + c0
        for sh in (8, 4, 2, 1):
            perm = lane ^ sh
            mp = _permute(m, perm)
            ep = _permute(e, perm)
            pred = (mp < m) | ((mp == m) & (ep < e))
            m = jnp.where(pred, mp, m)
            e = jnp.where(pred, ep, e)
        mres = jnp.where(lane == s, m, mres)
        eres = jnp.where(lane == s, e, eres)

    gwid = a * 2 + h
    mv[...] = mres
    ev[...] = eres
    pltpu.sync_copy(mv, hm.at[gwid])
    pltpu.sync_copy(ev, he.at[gwid])
    plsc.subcore_barrier()
    pltpu.sync_copy(hm.at[gwid ^ 1], pmv)
    pltpu.sync_copy(he.at[gwid ^ 1], pev)
    mp = pmv[...]
    ep = pev[...]
    pred = (mp < mres) | ((mp == mres) & (ep < eres))
    eres = jnp.where(pred, ep, eres)
    ev[...] = eres

    @pl.when(h == 0)
    def _():
        pltpu.sync_copy(ev.at[pl.ds(0, RPB)], out_hbm.at[pl.ds(r0, RPB)])


def kernel(x):
    out, _, _ = _argmin_rows_sc(x)
    return out


# speedup vs baseline: 3.3385x; 1.0505x over previous
import functools

import jax
import jax.numpy as jnp
from jax import lax
from jax.experimental import pallas as pl
from jax.experimental.pallas import tpu as pltpu
from jax.experimental.pallas import tpu_sc as plsc

NC, NS, L = 2, 16, 16
NW = NC * NS
ROWS, COLS = 128, 32768
SC_TROWS = 4
SC_ROWS = SC_TROWS * 8
RPB = 8
NH = NW // SC_TROWS
HW = COLS // NH
NCHUNK = 2
CW = HW // NCHUNK
FPC = CW // L
BIG = 2**31 - 1


def _permute(v, idx):
    dnums = lax.GatherDimensionNumbers(
        offset_dims=(), collapsed_slice_dims=(0,), start_index_map=(0,)
    )
    return lax.gather(
        v, idx[:, None], dnums, (1,),
        mode=lax.GatherScatterMode.PROMISE_IN_BOUNDS,
    )


_mesh = plsc.VectorSubcoreMesh(
    core_axis_name="c", subcore_axis_name="s", num_cores=NC, num_subcores=NS
)


@functools.partial(
    pl.kernel,
    out_type=(
        jax.ShapeDtypeStruct((SC_ROWS,), jnp.int32),
        jax.ShapeDtypeStruct((NW, L), jnp.float32),
        jax.ShapeDtypeStruct((NW, L), jnp.int32),
    ),
    mesh=_mesh,
    scratch_types=[
        pltpu.VMEM((2, RPB, CW), jnp.float32),
        pltpu.VMEM((L,), jnp.float32),
        pltpu.VMEM((L,), jnp.int32),
        pltpu.VMEM((L,), jnp.float32),
        pltpu.VMEM((L,), jnp.int32),
        pltpu.SemaphoreType.DMA,
        pltpu.SemaphoreType.DMA,
    ],
)
def _argmin_sc(x_hbm, out_hbm, hm, he, buf, mv, ev, pmv, pev, sem0, sem1):
    scid = lax.axis_index("c")
    sid = lax.axis_index("s")
    a = scid * (SC_TROWS // NC) + sid // NH
    h = sid % NH
    r0 = a * RPB
    c0 = h * HW
    lane = lax.iota(jnp.int32, 16)

    def chunk_src(c):
        return x_hbm.at[pl.ds(r0, RPB), pl.ds(c0 + c * CW, CW)]

    pltpu.async_copy(chunk_src(0), buf.at[0], sem0)
    pltpu.async_copy(chunk_src(1), buf.at[1], sem1)

    inf = jnp.full((L,), jnp.inf, jnp.float32)
    zero = jnp.zeros((L,), jnp.int32)

    def cbody(c, carry):
        ms, gs = list(carry[0]), list(carry[1])
        p = c % 2

        @pl.when(p == 0)
        def _():
            pltpu.make_async_copy(chunk_src(0), buf.at[0], sem0).wait()

        @pl.when(p == 1)
        def _():
            pltpu.make_async_copy(chunk_src(1), buf.at[1], sem1).wait()

        def fbody(f, carry2):
            ms2, gs2 = list(carry2[0]), list(carry2[1])
            vi = c * FPC + f
            for s in range(RPB):
                v = buf[p, s, pl.ds(f * L, L)]
                pred = v < ms2[s]
                ms2[s] = jnp.where(pred, v, ms2[s])
                gs2[s] = jnp.where(pred, jnp.full((L,), vi, jnp.int32),
                                   gs2[s])
            return tuple(ms2), tuple(gs2)

        return lax.fori_loop(0, FPC, fbody, (tuple(ms), tuple(gs)))

    ms, gs = lax.fori_loop(0, NCHUNK, cbody, ((inf,) * RPB, (zero,) * RPB))

    mres = jnp.full((L,), jnp.inf, jnp.float32)
    eres = jnp.full((L,), BIG, jnp.int32)
    for s in range(RPB):
        m = ms[s]
        e = (gs[s] * L + lane) + c0
        for sh in (8, 4, 2, 1):
            perm = lane ^ sh
            mp = _permute(m, perm)
            ep = _permute(e, perm)
            pred = (mp < m) | ((mp == m) & (ep < e))
            m = jnp.where(pred, mp, m)
            e = jnp.where(pred, ep, e)
        mres = jnp.where(lane == s, m, mres)
        eres = jnp.where(lane == s, e, eres)

    gwid = a * NH + h
    mv[...] = mres
    ev[...] = eres
    pltpu.sync_copy(mv, hm.at[gwid])
    pltpu.sync_copy(ev, he.at[gwid])
    plsc.subcore_barrier()

    @pl.when(h == 0)
    def _():
        best_m = mres
        best_e = eres
        for k in range(1, NH):
            pltpu.sync_copy(hm.at[gwid + k], pmv)
            pltpu.sync_copy(he.at[gwid + k], pev)
            mp = pmv[...]
            ep = pev[...]
            pred = (mp < best_m) | ((mp == best_m) & (ep < best_e))
            best_m = jnp.where(pred, mp, best_m)
            best_e = jnp.where(pred, ep, best_e)
        ev[...] = best_e
        pltpu.sync_copy(ev.at[pl.ds(0, RPB)], out_hbm.at[pl.ds(r0, RPB)])


def _tc_body(x_ref, o_ref):
    xb = x_ref[...]
    m = jnp.min(xb, axis=1, keepdims=True)
    col = lax.broadcasted_iota(jnp.int32, xb.shape, 1)
    big = jnp.full(xb.shape, BIG, jnp.int32)
    o_ref[...] = jnp.min(jnp.where(xb == m, col, big), axis=1)[None, None, :]


_tc_argmin = pl.pallas_call(
    _tc_body,
    grid=((ROWS - SC_ROWS) // RPB,),
    in_specs=[pl.BlockSpec((RPB, COLS), lambda i: (i + SC_TROWS, 0))],
    out_specs=pl.BlockSpec((1, 1, RPB), lambda i: (i, 0, 0)),
    out_shape=jax.ShapeDtypeStruct(
        ((ROWS - SC_ROWS) // RPB, 1, RPB), jnp.int32
    ),
)


def kernel(x):
    sc_out, _, _ = _argmin_sc(x)
    tc_out = _tc_argmin(x)
    return jnp.concatenate([sc_out, tc_out.reshape(-1)])
